# baseline (speedup 1.0000x reference)
import functools

import jax
import jax.numpy as jnp
from jax import lax
from jax.experimental import pallas as pl
from jax.experimental.pallas import tpu as pltpu
from jax.experimental.pallas import tpu_sc as plsc

B, C, D = 16384, 100, 64
ROW = C * D
NC, NS, L = 2, 16, 16
NW = NC * NS
BPW = B // NW
CHUNK = 4
NBUF = 4
NCHUNK = BPW // CHUNK
NGRP = NCHUNK // NBUF
NJ = ROW // L

_mesh = plsc.VectorSubcoreMesh(core_axis_name="c", subcore_axis_name="s")


@functools.partial(
    pl.kernel,
    mesh=_mesh,
    out_type=jax.ShapeDtypeStruct((B, ROW), jnp.float32),
    scratch_types=[
        pltpu.VMEM((ROW,), jnp.float32),
        pltpu.VMEM((NBUF, CHUNK, ROW), jnp.float32),
        pltpu.SemaphoreType.DMA((NBUF,)),
        pltpu.SemaphoreType.DMA((NBUF,)),
    ],
)
def _col_add(x_hbm, t_hbm, o_hbm, tbuf, bufs, sin, sout):
    wid = lax.axis_index("s") * NC + lax.axis_index("c")
    base = wid * BPW
    pltpu.sync_copy(t_hbm, tbuf)

    def start_in(chunk_idx, b):
        pltpu.async_copy(
            x_hbm.at[pl.ds(base + chunk_idx * CHUNK, CHUNK)],
            bufs.at[b], sin.at[b])

    def wait_in(b):
        pltpu.make_async_copy(
            x_hbm.at[pl.ds(base, CHUNK)], bufs.at[b], sin.at[b]).wait()

    def start_out(chunk_idx, b):
        pltpu.async_copy(
            bufs.at[b],
            o_hbm.at[pl.ds(base + chunk_idx * CHUNK, CHUNK)], sout.at[b])

    def wait_out(b):
        pltpu.make_async_copy(
            bufs.at[b], o_hbm.at[pl.ds(base, CHUNK)], sout.at[b]).wait()

    def compute(b):
        def j_body(j, c2):
            sl = pl.ds(j * L, L)
            t = tbuf[sl]
            for r in range(CHUNK):
                bufs[b, r, sl] += t
            return c2
        lax.fori_loop(0, NJ, j_body, 0, unroll=2)

    start_in(0, 0)
    start_in(1, 1)

    def group(g, carry):
        for b in range(NBUF):
            i = g * NBUF + b
            bp = (b + 2) % NBUF

            @pl.when(i + 2 < NCHUNK)
            def _():
                @pl.when(i >= 2)
                def _():
                    wait_out(bp)
                start_in(i + 2, bp)

            wait_in(b)
            compute(b)
            start_out(i, b)
        return carry

    lax.fori_loop(0, NGRP, group, 0)
    for b in range(NBUF):
        wait_out(b)


def kernel(inputs, table):
    out = _col_add(inputs.reshape(B, ROW), table.reshape(ROW))
    return out.reshape(B, C, D)


# speedup vs baseline: 1.5714x; 1.5714x over previous
import functools

import jax
import jax.numpy as jnp
from jax import lax
from jax.experimental import pallas as pl
from jax.experimental.pallas import tpu as pltpu
from jax.experimental.pallas import tpu_sc as plsc

B, C, D = 16384, 100, 64
ROW = C * D
NC, NS, L = 2, 16, 16
NW = NC * NS
BPW = B // NW
CHUNK = 4
NBUF = 4
NCHUNK = BPW // CHUNK
NGRP = NCHUNK // NBUF
NJ = ROW // L

_mesh = plsc.VectorSubcoreMesh(core_axis_name="c", subcore_axis_name="s")


@functools.partial(
    pl.kernel,
    mesh=_mesh,
    out_type=jax.ShapeDtypeStruct((B, ROW), jnp.float32),
    scratch_types=[
        pltpu.VMEM((ROW,), jnp.float32),
        pltpu.VMEM((NBUF, CHUNK, ROW), jnp.float32),
        pltpu.SemaphoreType.DMA((NBUF,)),
        pltpu.SemaphoreType.DMA((NBUF,)),
    ],
)
def _col_add(x_hbm, t_hbm, o_hbm, tbuf, bufs, sin, sout):
    wid = lax.axis_index("s") * NC + lax.axis_index("c")
    base = wid * BPW
    pltpu.sync_copy(t_hbm, tbuf)

    def start_in(chunk_idx, b):
        pltpu.async_copy(
            x_hbm.at[pl.ds(base + chunk_idx * CHUNK, CHUNK)],
            bufs.at[b], sin.at[b])

    def wait_in(b):
        pltpu.make_async_copy(
            x_hbm.at[pl.ds(base, CHUNK)], bufs.at[b], sin.at[b]).wait()

    def start_out(chunk_idx, b):
        pltpu.async_copy(
            bufs.at[b],
            o_hbm.at[pl.ds(base + chunk_idx * CHUNK, CHUNK)], sout.at[b])

    def wait_out(b):
        pltpu.make_async_copy(
            bufs.at[b], o_hbm.at[pl.ds(base, CHUNK)], sout.at[b]).wait()

    def compute(b):
        @plsc.parallel_loop(0, NJ, unroll=4)
        def _(j):
            sl = pl.ds(j * L, L)
            t = tbuf[sl]
            for r in range(CHUNK):
                bufs[b, r, sl] += t

    start_in(0, 0)
    start_in(1, 1)

    def group(g, carry):
        for b in range(NBUF):
            i = g * NBUF + b
            bp = (b + 2) % NBUF

            @pl.when(i + 2 < NCHUNK)
            def _():
                @pl.when(i >= 2)
                def _():
                    wait_out(bp)
                start_in(i + 2, bp)

            wait_in(b)
            compute(b)
            start_out(i, b)
        return carry

    lax.fori_loop(0, NGRP, group, 0)
    for b in range(NBUF):
        wait_out(b)


def kernel(inputs, table):
    out = _col_add(inputs.reshape(B, ROW), table.reshape(ROW))
    return out.reshape(B, C, D)
